# baseline (speedup 1.0000x reference)
import functools

import jax
import jax.numpy as jnp
from jax import lax
from jax.experimental import pallas as pl
from jax.experimental.pallas import tpu as pltpu
from jax.experimental.pallas import tpu_sc as plsc

DEPTH = 32
B = 4
H = 512
W = 512
NBLK = 8
ROWS = H // NBLK
HROWS = ROWS // 2
CHUNK = ROWS * W
LANES = 16
NSLICE = W // LANES


def _compute_pair(x_v, xbase, bufa, bufb, d0, d1):
    one = jnp.float32(1.0)
    zero = jnp.float32(0.0)

    def row(r, _):
        xrow = xbase + r * W
        for c in range(NSLICE):
            x = x_v[pl.ds(xrow + c * LANES, LANES)]
            bufa[r, pl.ds(c * LANES, LANES)] = jnp.where(x == d0, one, zero)
            bufb[r, pl.ds(c * LANES, LANES)] = jnp.where(x == d1, one, zero)
        return 0

    lax.fori_loop(0, HROWS, row, 0, unroll=False)


def _sc_body(x_hbm, out_hbm, x_v, ba0, bb0, ba1, bb1, sa0, sb0, sa1, sb1):
    nc = 2
    wid = lax.axis_index("s") * nc + lax.axis_index("c")
    b = wid // NBLK
    hblk = wid % NBLK
    row0 = hblk * ROWS

    pltpu.sync_copy(x_hbm.at[b, hblk], x_v)

    bufs = ((ba0, bb0, sa0, sb0), (ba1, bb1, sa1, sb1))

    def depth_pair(i, _):
        d0 = 2 * i
        d1 = d0 + 1
        for half in (0, 1):
            bufa, bufb, sema, semb = bufs[half]
            dsta = out_hbm.at[b, d0, pl.ds(row0 + half * HROWS, HROWS)]
            dstb = out_hbm.at[b, d1, pl.ds(row0 + half * HROWS, HROWS)]

            @pl.when(i > 0)
            def _():
                pltpu.make_async_copy(bufa, dsta, sema).wait()
                pltpu.make_async_copy(bufb, dstb, semb).wait()

            _compute_pair(x_v, half * HROWS * W, bufa, bufb, d0, d1)
            pltpu.make_async_copy(bufa, dsta, sema).start()
            pltpu.make_async_copy(bufb, dstb, semb).start()
        return 0

    lax.fori_loop(0, DEPTH // 2, depth_pair, 0, unroll=False)

    for half in (0, 1):
        bufa, bufb, sema, semb = bufs[half]
        dst = out_hbm.at[b, 0, pl.ds(row0 + half * HROWS, HROWS)]
        pltpu.make_async_copy(bufa, dst, sema).wait()
        pltpu.make_async_copy(bufb, dst, semb).wait()


def kernel(rank, X_in, ones):
    x = X_in.reshape(B, NBLK, CHUNK)
    mesh = plsc.VectorSubcoreMesh(core_axis_name="c", subcore_axis_name="s")
    run = functools.partial(
        pl.kernel,
        mesh=mesh,
        compiler_params=pltpu.CompilerParams(use_tc_tiling_on_sc=False),
        out_type=jax.ShapeDtypeStruct((B, DEPTH, H, W), jnp.float32),
        scratch_types=[
            pltpu.VMEM((CHUNK,), jnp.int32),
            pltpu.VMEM((HROWS, W), jnp.float32),
            pltpu.VMEM((HROWS, W), jnp.float32),
            pltpu.VMEM((HROWS, W), jnp.float32),
            pltpu.VMEM((HROWS, W), jnp.float32),
            pltpu.SemaphoreType.DMA,
            pltpu.SemaphoreType.DMA,
            pltpu.SemaphoreType.DMA,
            pltpu.SemaphoreType.DMA,
        ],
    )(_sc_body)
    return run(x)


# speedup vs baseline: 2.6368x; 2.6368x over previous
import functools

import jax
import jax.numpy as jnp
from jax import lax
from jax.experimental import pallas as pl
from jax.experimental.pallas import tpu as pltpu
from jax.experimental.pallas import tpu_sc as plsc

DEPTH = 32
B = 4
H = 512
W = 512
NW = 32
CHUNK = B * H * W // NW
HALF = CHUNK // 2
LANES = 16
UNROLL = 4
HB = 64


def _shift_half(x_v, xoff, buf):
    one = jnp.int32(1)

    def body(j, _):
        base = j * (LANES * UNROLL)
        for u in range(UNROLL):
            off = base + u * LANES
            x = x_v[pl.ds(xoff + off, LANES)]
            buf[pl.ds(off, LANES)] = one << x
        return 0

    lax.fori_loop(0, HALF // (LANES * UNROLL), body, 0, unroll=False)


def _sc_encode(x_hbm, mask_hbm, x_v, buf0, buf1, sem0, sem1):
    nc = 2
    wid = lax.axis_index("s") * nc + lax.axis_index("c")

    pltpu.sync_copy(x_hbm.at[wid], x_v)

    bufs = ((buf0, sem0), (buf1, sem1))
    for half in (0, 1):
        buf, sem = bufs[half]
        _shift_half(x_v, half * HALF, buf)
        pltpu.make_async_copy(buf, mask_hbm.at[wid, half], sem).start()
    for half in (0, 1):
        buf, sem = bufs[half]
        pltpu.make_async_copy(buf, mask_hbm.at[wid, half], sem).wait()


def _tc_expand(mask_ref, out_ref):
    m = mask_ref[...]
    d = jax.lax.broadcasted_iota(jnp.int32, (1, DEPTH, HB, W), 1)
    bit = jax.lax.shift_right_logical(m, d) & jnp.int32(1)
    out_ref[...] = bit.astype(jnp.float32)


def kernel(rank, X_in, ones):
    x = X_in.reshape(NW, CHUNK)
    mesh = plsc.VectorSubcoreMesh(core_axis_name="c", subcore_axis_name="s")
    encode = functools.partial(
        pl.kernel,
        mesh=mesh,
        out_type=jax.ShapeDtypeStruct((NW, 2, HALF), jnp.int32),
        scratch_types=[
            pltpu.VMEM((CHUNK,), jnp.int32),
            pltpu.VMEM((HALF,), jnp.int32),
            pltpu.VMEM((HALF,), jnp.int32),
            pltpu.SemaphoreType.DMA,
            pltpu.SemaphoreType.DMA,
        ],
    )(_sc_encode)
    mask = encode(x).reshape(B, 1, H, W)

    out = pl.pallas_call(
        _tc_expand,
        grid=(B, H // HB),
        in_specs=[pl.BlockSpec((1, 1, HB, W), lambda b, h: (b, 0, h, 0))],
        out_specs=pl.BlockSpec((1, DEPTH, HB, W), lambda b, h: (b, 0, h, 0)),
        out_shape=jax.ShapeDtypeStruct((B, DEPTH, H, W), jnp.float32),
    )(mask)
    return out


# speedup vs baseline: 2.7362x; 1.0377x over previous
import functools

import jax
import jax.numpy as jnp
from jax import lax
from jax.experimental import pallas as pl
from jax.experimental.pallas import tpu as pltpu
from jax.experimental.pallas import tpu_sc as plsc

DEPTH = 32
B = 4
H = 512
W = 512
NW = 32
CHUNK = B * H * W // NW
HALF = CHUNK // 2
LANES = 16
UNROLL = 4
HB = 128


def _shift_half(x_v, xoff, buf):
    one = jnp.int32(1)

    def body(j, _):
        base = j * (LANES * UNROLL)
        for u in range(UNROLL):
            off = base + u * LANES
            x = x_v[pl.ds(xoff + off, LANES)]
            buf[pl.ds(off, LANES)] = one << x
        return 0

    lax.fori_loop(0, HALF // (LANES * UNROLL), body, 0, unroll=False)


def _sc_encode(x_hbm, mask_hbm, x_v, buf0, buf1, sem0, sem1, semi0, semi1):
    nc = 2
    wid = lax.axis_index("s") * nc + lax.axis_index("c")

    in0 = pltpu.make_async_copy(x_hbm.at[wid, 0], x_v.at[pl.ds(0, HALF)], semi0)
    in1 = pltpu.make_async_copy(
        x_hbm.at[wid, 1], x_v.at[pl.ds(HALF, HALF)], semi1)
    in0.start()
    in1.start()

    in0.wait()
    _shift_half(x_v, 0, buf0)
    out0 = pltpu.make_async_copy(buf0, mask_hbm.at[wid, 0], sem0)
    out0.start()

    in1.wait()
    _shift_half(x_v, HALF, buf1)
    out1 = pltpu.make_async_copy(buf1, mask_hbm.at[wid, 1], sem1)
    out1.start()

    out0.wait()
    out1.wait()


def _tc_expand(mask_ref, out_ref):
    m = mask_ref[...]
    d = jax.lax.broadcasted_iota(jnp.int32, (1, DEPTH, HB, W), 1)
    bit = jax.lax.shift_right_logical(m, d) & jnp.int32(1)
    out_ref[...] = bit.astype(jnp.float32)


def kernel(rank, X_in, ones):
    x = X_in.reshape(NW, 2, HALF)
    mesh = plsc.VectorSubcoreMesh(core_axis_name="c", subcore_axis_name="s")
    encode = functools.partial(
        pl.kernel,
        mesh=mesh,
        out_type=jax.ShapeDtypeStruct((NW, 2, HALF), jnp.int32),
        scratch_types=[
            pltpu.VMEM((CHUNK,), jnp.int32),
            pltpu.VMEM((HALF,), jnp.int32),
            pltpu.VMEM((HALF,), jnp.int32),
            pltpu.SemaphoreType.DMA,
            pltpu.SemaphoreType.DMA,
            pltpu.SemaphoreType.DMA,
            pltpu.SemaphoreType.DMA,
        ],
    )(_sc_encode)
    mask = encode(x).reshape(B, 1, H, W)

    out = pl.pallas_call(
        _tc_expand,
        grid=(B, H // HB),
        in_specs=[pl.BlockSpec((1, 1, HB, W), lambda b, h: (b, 0, h, 0))],
        out_specs=pl.BlockSpec((1, DEPTH, HB, W), lambda b, h: (b, 0, h, 0)),
        out_shape=jax.ShapeDtypeStruct((B, DEPTH, H, W), jnp.float32),
    )(mask)
    return out
